# baseline (speedup 1.0000x reference)
import functools

import jax
import jax.numpy as jnp
from jax import lax
from jax.experimental import pallas as pl
from jax.experimental.pallas import tpu as pltpu
from jax.experimental.pallas import tpu_sc as plsc

D = 32
NW = 32
L = 16

TC_CHUNK = 800
TC_NBUF = 2

G_NBUF = 4


def _wid():
    return lax.axis_index("s") * 2 + lax.axis_index("c")


@functools.lru_cache(maxsize=None)
def _build_transpose(vocab: int):
    n_chunks = vocab // TC_CHUNK
    rounds = (n_chunks + NW * TC_NBUF - 1) // (NW * TC_NBUF)
    mesh = plsc.VectorSubcoreMesh(core_axis_name="c", subcore_axis_name="s")

    @functools.partial(
        pl.kernel,
        mesh=mesh,
        out_type=jax.ShapeDtypeStruct((vocab, D), jnp.float32),
        scratch_types=[
            pltpu.VMEM((TC_NBUF, D, TC_CHUNK), jnp.float32),
            pltpu.VMEM((TC_NBUF, TC_CHUNK, D), jnp.float32),
            pltpu.SemaphoreType.DMA,
            pltpu.SemaphoreType.DMA,
            pltpu.SemaphoreType.DMA,
            pltpu.SemaphoreType.DMA,
        ],
        compiler_params=pltpu.CompilerParams(use_tc_tiling_on_sc=False, needs_layout_passes=False),
    )
    def transpose_kernel(tableT_hbm, out_hbm, in_v, out_v, rs0, rs1, ws0, ws1):
        rs = [rs0, rs1]
        ws = [ws0, ws1]
        w = _wid()
        lane = lax.iota(jnp.int32, L)

        def round_(r, carry):
            cs = [w + NW * (r * TC_NBUF + b) for b in range(TC_NBUF)]
            for b in range(TC_NBUF):
                @pl.when(cs[b] < n_chunks)
                def _(b=b):
                    pltpu.async_copy(
                        tableT_hbm.at[:, pl.ds(cs[b] * TC_CHUNK, TC_CHUNK)],
                        in_v.at[b], rs[b])
            for b in range(TC_NBUF):
                @pl.when(cs[b] < n_chunks)
                def _(b=b):
                    pltpu.make_async_copy(
                        tableT_hbm.at[:, pl.ds(cs[b] * TC_CHUNK, TC_CHUNK)],
                        in_v.at[b], rs[b]).wait()

                    def col_group(g, carry2):
                        rows = g * L + lane
                        for d in range(D):
                            vec = in_v[b, d, pl.ds(g * L, L)]
                            plsc.store_scatter(
                                out_v.at[b], [rows, jnp.full((L,), d, jnp.int32)],
                                vec)
                        return carry2

                    lax.fori_loop(0, TC_CHUNK // L, col_group, 0)
                    pltpu.async_copy(
                        out_v.at[b],
                        out_hbm.at[pl.ds(cs[b] * TC_CHUNK, TC_CHUNK)], ws[b])
            for b in range(TC_NBUF):
                @pl.when(cs[b] < n_chunks)
                def _(b=b):
                    pltpu.make_async_copy(
                        out_v.at[b],
                        out_hbm.at[pl.ds(cs[b] * TC_CHUNK, TC_CHUNK)],
                        ws[b]).wait()
            return carry

        lax.fori_loop(0, rounds, round_, 0)

    return transpose_kernel


@functools.lru_cache(maxsize=None)
def _build_gather(batch: int, hist: int):
    nbt = batch // 128
    n_units = hist * nbt
    units_w = n_units // NW
    rounds = units_w // G_NBUF
    mesh = plsc.VectorSubcoreMesh(core_axis_name="c", subcore_axis_name="s")

    @functools.partial(
        pl.kernel,
        mesh=mesh,
        out_type=jax.ShapeDtypeStruct((hist, D // 8, nbt, 8 * 128),
                                      jnp.float32),
        scratch_types=[
            pltpu.VMEM((G_NBUF, 128), jnp.int32),
            pltpu.VMEM((G_NBUF, 128, D), jnp.float32),
            pltpu.VMEM((G_NBUF, D // 8, 8 * 128), jnp.float32),
            pltpu.SemaphoreType.DMA,
            pltpu.SemaphoreType.DMA,
            pltpu.SemaphoreType.DMA,
            pltpu.SemaphoreType.DMA,
            pltpu.SemaphoreType.DMA,
            pltpu.SemaphoreType.DMA,
            pltpu.SemaphoreType.DMA,
            pltpu.SemaphoreType.DMA,
            pltpu.SemaphoreType.DMA,
            pltpu.SemaphoreType.DMA,
            pltpu.SemaphoreType.DMA,
            pltpu.SemaphoreType.DMA,
        ],
        compiler_params=pltpu.CompilerParams(use_tc_tiling_on_sc=False, needs_layout_passes=False),
    )
    def gather_kernel(idxT_hbm, table_hbm, out_hbm, idx_v, rows_v, out_v,
                      *sems):
        gs = sems[:G_NBUF]
        ws = sems[G_NBUF:2 * G_NBUF]
        is_ = sems[2 * G_NBUF:3 * G_NBUF]
        w = _wid()
        u0 = w * units_w
        lane = lax.iota(jnp.int32, L)

        def round_(r, carry):
            us = [u0 + r * G_NBUF + b for b in range(G_NBUF)]
            hs = [u // nbt for u in us]
            bts = [u % nbt for u in us]
            for b in range(G_NBUF):
                pltpu.make_async_copy(
                    idxT_hbm.at[hs[b], pl.ds(bts[b] * 128, 128)],
                    idx_v.at[b], is_[b]).start()
            for b in range(G_NBUF):
                pltpu.make_async_copy(
                    idxT_hbm.at[hs[b], pl.ds(bts[b] * 128, 128)],
                    idx_v.at[b], is_[b]).wait()
                pltpu.async_copy(table_hbm.at[idx_v.at[b]],
                                 rows_v.at[b], gs[b])
            for b in range(G_NBUF):
                pltpu.make_async_copy(table_hbm.at[idx_v.at[b]],
                                      rows_v.at[b], gs[b]).wait()

                def dim(d, carry2):
                    dt = d // 8
                    di = d % 8
                    dvec = jnp.full((L,), d, jnp.int32)
                    for jg in range(128 // L):
                        vec = plsc.load_gather(
                            rows_v.at[b], [jg * L + lane, dvec])
                        out_v[b, dt, pl.ds(di * 128 + jg * L, L)] = vec
                    return carry2

                lax.fori_loop(0, D, dim, 0)
                pltpu.async_copy(
                    out_v.at[b], out_hbm.at[hs[b]].at[:, bts[b]], ws[b])
            for b in range(G_NBUF):
                pltpu.make_async_copy(
                    out_v.at[b], out_hbm.at[hs[b]].at[:, bts[b]],
                    ws[b]).wait()
            return carry

        lax.fori_loop(0, rounds, round_, 0)

    return gather_kernel


def kernel(inputs, table):
    batch, hist = inputs.shape
    vocab = table.shape[0]
    table_rm = _build_transpose(vocab)(table.T)
    raw = _build_gather(batch, hist)(inputs.T, table_rm)
    o5 = raw.reshape(hist, D // 8, batch // 128, 8, 128)
    return o5.transpose(2, 4, 0, 1, 3).reshape(batch, hist, D)


# speedup vs baseline: 3.7913x; 3.7913x over previous
import functools

import jax
import jax.numpy as jnp
from jax import lax
from jax.experimental import pallas as pl
from jax.experimental.pallas import tpu as pltpu
from jax.experimental.pallas import tpu_sc as plsc

D = 32
NW = 32
NB = 16
NBUF = 2


@functools.lru_cache(maxsize=None)
def _build(batch: int, hist: int):
    rows_w = batch // NW
    chunks_w = rows_w // NB
    rounds = chunks_w // NBUF
    mesh = plsc.VectorSubcoreMesh(core_axis_name="c", subcore_axis_name="s")

    @functools.partial(
        pl.kernel,
        mesh=mesh,
        out_type=jax.ShapeDtypeStruct((batch, hist, D), jnp.float32),
        scratch_types=[
            pltpu.VMEM((NBUF, NB, hist), jnp.int32),
            pltpu.VMEM((NBUF, NB, hist, D), jnp.float32),
            pltpu.SemaphoreType.DMA,
            pltpu.SemaphoreType.DMA,
            pltpu.SemaphoreType.DMA,
            pltpu.SemaphoreType.DMA,
        ],
        compiler_params=pltpu.CompilerParams(use_tc_tiling_on_sc=False),
    )
    def gather_kernel(idx_hbm, table_hbm, out_hbm, idx_v, rows_v,
                      gs0, gs1, ws0, ws1):
        gs = [gs0, gs1]
        ws = [ws0, ws1]
        wid = lax.axis_index("s") * 2 + lax.axis_index("c")
        r0 = wid * rows_w

        def round_(r, carry):
            bs = [r0 + (r * NBUF + b) * NB for b in range(NBUF)]
            ghandles = []
            for b in range(NBUF):
                pltpu.sync_copy(idx_hbm.at[pl.ds(bs[b], NB)], idx_v.at[b])
                for j in range(NB):
                    ghandles.append(pltpu.async_copy(
                        table_hbm.at[idx_v.at[b].at[j]],
                        rows_v.at[b].at[j], gs[b]))
            whandles = []
            for b in range(NBUF):
                for j in range(NB):
                    ghandles[b * NB + j].wait()
                whandles.append(pltpu.async_copy(
                    rows_v.at[b], out_hbm.at[pl.ds(bs[b], NB)], ws[b]))
            for b in range(NBUF):
                whandles[b].wait()
            return carry

        lax.fori_loop(0, rounds, round_, 0)

    return gather_kernel


def kernel(inputs, table):
    b, h = inputs.shape
    return _build(b, h)(inputs, table)
